# Initial kernel scaffold; baseline (speedup 1.0000x reference)
import jax
import jax.numpy as jnp
from jax.experimental import pallas as pl


def kernel(x, edge_index, batch, emb, W1, b1, W2, b2, gamma, beta):
    raise NotImplementedError("write your pallas kernel here")


# speedup vs baseline: 3.0072x; 3.0072x over previous
import functools

import jax
import jax.numpy as jnp
from jax import lax
from jax.experimental import pallas as pl
from jax.experimental.pallas import tpu as pltpu
from jax.experimental.pallas import tpu_sc as plsc

N = 10000
D = 128
G = 256
L = 3
NP = 10240
NGROUPS = NP // 128


def _mesh():
    return plsc.VectorSubcoreMesh(core_axis_name="c", subcore_axis_name="s")


def _emb_body(idx_hbm, emb_hbm, h_hbm, idx_v, rows_v, sem):
    c = lax.axis_index("c")
    s = lax.axis_index("s")
    w = s * 2 + c

    def do(j, carry):
        gid = w + 32 * j

        @pl.when(gid < NGROUPS)
        def _():
            pltpu.sync_copy(idx_hbm.at[gid], idx_v)
            pltpu.async_copy(emb_hbm.at[idx_v], rows_v, sem).wait()
            pltpu.sync_copy(rows_v, h_hbm.at[pl.ds(gid * 128, 128), :])

        return carry

    lax.fori_loop(0, (NGROUPS + 31) // 32, do, 0)


@functools.lru_cache(maxsize=None)
def _emb_call():
    return pl.kernel(
        _emb_body,
        out_type=jax.ShapeDtypeStruct((NP, D), jnp.float32),
        mesh=_mesh(),
        scratch_types=[
            pltpu.VMEM((128,), jnp.int32),
            pltpu.VMEM((128, D), jnp.float32),
            pltpu.SemaphoreType.DMA,
        ],
    )


def _segsum_body(gpw, h_hbm, src_hbm, dst_hbm, agg_hbm,
                 src_v, dst_v, rows_v, zed_v, agg_sp, sem):
    c = lax.axis_index("c")
    s = lax.axis_index("s")
    row0 = s * (NP // 16)

    def zstore(i, carry):
        zed_v[i // 8, pl.ds((i % 8) * 16, 16)] = jnp.zeros((16,), jnp.float32)
        return carry

    lax.fori_loop(0, 64 * 8, zstore, 0)

    def zcp(i, carry):
        pltpu.sync_copy(zed_v, agg_sp.at[pl.ds(row0 + i * 64, 64), :])
        return carry

    lax.fori_loop(0, (NP // 16) // 64, zcp, 0)

    plsc.subcore_barrier()

    g0 = (c * 16 + s) * gpw
    pltpu.sync_copy(src_hbm.at[pl.ds(g0, gpw), :], src_v)
    pltpu.sync_copy(dst_hbm.at[pl.ds(g0, gpw), :], dst_v)

    def edge(j, carry):
        pltpu.async_copy(h_hbm.at[src_v.at[j]], rows_v, sem).wait()
        pltpu.sync_copy(rows_v, agg_sp.at[dst_v.at[j]], add=True)
        return carry

    lax.fori_loop(0, gpw, edge, 0)

    plsc.subcore_barrier()
    pltpu.sync_copy(agg_sp.at[pl.ds(row0, NP // 16), :],
                    agg_hbm.at[c, pl.ds(row0, NP // 16), :])


@functools.lru_cache(maxsize=None)
def _segsum_call(gpw):
    return pl.kernel(
        functools.partial(_segsum_body, gpw),
        out_type=jax.ShapeDtypeStruct((2, NP, D), jnp.float32),
        mesh=_mesh(),
        scratch_types=[
            pltpu.VMEM((gpw, 128), jnp.int32),
            pltpu.VMEM((gpw, 128), jnp.int32),
            pltpu.VMEM((128, D), jnp.float32),
            pltpu.VMEM((64, D), jnp.float32),
            pltpu.VMEM_SHARED((NP, D), jnp.float32),
            pltpu.SemaphoreType.DMA,
        ],
    )


def _mlp_body(h_ref, agg_ref, w1_ref, b1_ref, w2_ref, b2_ref,
              ga_ref, be_ref, o_ref):
    z = h_ref[...] + agg_ref[0] + agg_ref[1]
    z = jnp.maximum(
        jnp.dot(z, w1_ref[...], preferred_element_type=jnp.float32)
        + b1_ref[...], 0.0)
    z = jnp.maximum(
        jnp.dot(z, w2_ref[...], preferred_element_type=jnp.float32)
        + b2_ref[...], 0.0)
    msk = (lax.broadcasted_iota(jnp.int32, (NP, 1), 0) < N).astype(jnp.float32)
    zm = z * msk
    mean = jnp.sum(zm, axis=0, keepdims=True) * (1.0 / N)
    diff = (z - mean) * msk
    var = jnp.sum(diff * diff, axis=0, keepdims=True) * (1.0 / N)
    o_ref[...] = ((z - mean) * lax.rsqrt(var + 1e-5) * ga_ref[...]
                  + be_ref[...])


def _mlp(h, agg, w1, b1, w2, b2, ga, be):
    return pl.pallas_call(
        _mlp_body,
        out_shape=jax.ShapeDtypeStruct((NP, D), jnp.float32),
    )(h, agg, w1, b1, w2, b2, ga, be)


def _pool_body(z1_hbm, z2_hbm, z3_hbm, bt_hbm, out_hbm,
               bt_v, acc_v, b1_v, b2_v, b3_v, sem):
    c = lax.axis_index("c")
    s = lax.axis_index("s")
    t = s * 2 + c
    glo = t * 8
    ghi = glo + 8

    pltpu.sync_copy(bt_hbm, bt_v.at[pl.ds(0, NP)])
    bt_v[pl.ds(NP, 16)] = jnp.full((16,), 2 ** 30, jnp.int32)

    def lower_bound(val):
        def body(i, st):
            lo, hi = st
            mid = (lo + hi) // 2
            v = bt_v[pl.ds(mid, 16)][0]
            pred = v < val
            return (jnp.where(pred, mid + 1, lo), jnp.where(pred, hi, mid))

        return lax.fori_loop(0, 14, body, (jnp.int32(0), jnp.int32(NP)))[0]

    s0 = lower_bound(glo)
    e0 = lower_bound(ghi)

    neg = jnp.full((16,), -jnp.inf, dtype=jnp.float32)

    def ini(i, carry):
        acc_v[i // 24, pl.ds((i % 24) * 16, 16)] = neg
        return carry

    lax.fori_loop(0, 8 * 24, ini, 0)

    b0 = s0 // 16
    nb = (e0 + 15) // 16 - b0

    def blk(b, carry):
        r0 = pl.multiple_of((b0 + b) * 16, 16)
        pltpu.sync_copy(z1_hbm.at[pl.ds(r0, 16), :], b1_v)
        pltpu.sync_copy(z2_hbm.at[pl.ds(r0, 16), :], b2_v)
        pltpu.sync_copy(z3_hbm.at[pl.ds(r0, 16), :], b3_v)
        bv = bt_v[pl.ds(r0, 16)]

        for k in range(16):
            r = r0 + k

            @pl.when((r >= s0) & (r < e0))
            def _(k=k):
                g = bv[k] - glo

                def feat(j, c3, buf, base):
                    off = base + j * 16
                    acc_v[g, pl.ds(off, 16)] = jnp.maximum(
                        acc_v[g, pl.ds(off, 16)], buf[k, pl.ds(j * 16, 16)])
                    return c3

                lax.fori_loop(0, 8, functools.partial(
                    feat, buf=b1_v, base=0), 0)
                lax.fori_loop(0, 8, functools.partial(
                    feat, buf=b2_v, base=128), 0)
                lax.fori_loop(0, 8, functools.partial(
                    feat, buf=b3_v, base=256), 0)

        return carry

    lax.fori_loop(0, nb, blk, 0)

    pltpu.sync_copy(acc_v, out_hbm.at[pl.ds(glo, 8), :])


@functools.lru_cache(maxsize=None)
def _pool_call():
    return pl.kernel(
        _pool_body,
        out_type=jax.ShapeDtypeStruct((G, L * D), jnp.float32),
        mesh=_mesh(),
        scratch_types=[
            pltpu.VMEM((NP + 16,), jnp.int32),
            pltpu.VMEM((8, L * D), jnp.float32),
            pltpu.VMEM((16, D), jnp.float32),
            pltpu.VMEM((16, D), jnp.float32),
            pltpu.VMEM((16, D), jnp.float32),
            pltpu.SemaphoreType.DMA,
        ],
    )


def kernel(x, edge_index, batch, emb, W1, b1, W2, b2, gamma, beta):
    idx = x[:, 0].astype(jnp.int32)
    n = idx.shape[0]
    idx_p = jnp.pad(idx, (0, NP - n)).reshape(NGROUPS, 128)
    bt_p = jnp.pad(batch.astype(jnp.int32), (0, NP - n), constant_values=G)

    src = edge_index[0].astype(jnp.int32)
    dst = edge_index[1].astype(jnp.int32)
    e = src.shape[0]
    egroups = -(-e // 128)
    gpw = -(-egroups // 32)
    gpw = -(-gpw // 8) * 8
    ep = gpw * 32 * 128
    src_p = jnp.pad(src, (0, ep - e)).reshape(gpw * 32, 128)
    dst_p = jnp.pad(dst, (0, ep - e), constant_values=NP - 1).reshape(
        gpw * 32, 128)

    h = _emb_call()(idx_p, emb)

    outs = []
    for i in range(L):
        agg = _segsum_call(gpw)(h, src_p, dst_p)
        h = _mlp(h, agg, W1[i], b1[i].reshape(1, D), W2[i], b2[i].reshape(1, D),
                 gamma[i].reshape(1, D), beta[i].reshape(1, D))
        outs.append(h)

    return _pool_call()(outs[0], outs[1], outs[2], bt_p)


# speedup vs baseline: 6.4075x; 2.1307x over previous
import functools

import jax
import jax.numpy as jnp
from jax import lax
from jax.experimental import pallas as pl
from jax.experimental.pallas import tpu as pltpu
from jax.experimental.pallas import tpu_sc as plsc

N = 10000
D = 128
G = 256
L = 3
NP = 10240
NGROUPS = NP // 128


def _mesh():
    return plsc.VectorSubcoreMesh(core_axis_name="c", subcore_axis_name="s")


def _emb_body(idx_hbm, emb_hbm, h_hbm, idx_v, rows_v, sem):
    c = lax.axis_index("c")
    s = lax.axis_index("s")
    w = s * 2 + c

    def do(j, carry):
        gid = w + 32 * j

        @pl.when(gid < NGROUPS)
        def _():
            pltpu.sync_copy(idx_hbm.at[gid], idx_v)
            pltpu.async_copy(emb_hbm.at[idx_v], rows_v, sem).wait()
            pltpu.sync_copy(rows_v, h_hbm.at[pl.ds(gid * 128, 128), :])

        return carry

    lax.fori_loop(0, (NGROUPS + 31) // 32, do, 0)


@functools.lru_cache(maxsize=None)
def _emb_call():
    return pl.kernel(
        _emb_body,
        out_type=jax.ShapeDtypeStruct((NP, D), jnp.float32),
        mesh=_mesh(),
        scratch_types=[
            pltpu.VMEM((128,), jnp.int32),
            pltpu.VMEM((128, D), jnp.float32),
            pltpu.SemaphoreType.DMA,
        ],
    )


def _segsum_body(gpw, h_hbm, src_hbm, dst_hbm, agg_hbm,
                 src_v, dst_v, rows_v, zed_v, h_sp, agg_sp, sem):
    c = lax.axis_index("c")
    s = lax.axis_index("s")
    col0 = c * 64
    row0 = s * (NP // 16)

    pltpu.sync_copy(h_hbm.at[pl.ds(row0, NP // 16), pl.ds(col0, 64)],
                    h_sp.at[pl.ds(row0, NP // 16), :])

    def zstore(i, carry):
        zed_v[i // 4, pl.ds((i % 4) * 16, 16)] = jnp.zeros((16,), jnp.float32)
        return carry

    lax.fori_loop(0, 64 * 4, zstore, 0)

    def zcp(i, carry):
        pltpu.sync_copy(zed_v, agg_sp.at[pl.ds(row0 + i * 64, 64), :])
        return carry

    lax.fori_loop(0, (NP // 16) // 64, zcp, 0)

    plsc.subcore_barrier()

    g0 = s * gpw

    def chunk(ci, carry):
        cg = g0 + ci * 32
        pltpu.sync_copy(src_hbm.at[pl.ds(cg, 32), :], src_v)
        pltpu.sync_copy(dst_hbm.at[pl.ds(cg, 32), :], dst_v)

        def edge(j, carry2):
            pltpu.async_copy(h_sp.at[src_v.at[j]], rows_v, sem).wait()
            pltpu.sync_copy(rows_v, agg_sp.at[dst_v.at[j]], add=True)
            return carry2

        lax.fori_loop(0, 32, edge, 0)
        return carry

    lax.fori_loop(0, gpw // 32, chunk, 0)

    plsc.subcore_barrier()
    pltpu.sync_copy(agg_sp.at[pl.ds(row0, NP // 16), :],
                    agg_hbm.at[pl.ds(row0, NP // 16), pl.ds(col0, 64)])


@functools.lru_cache(maxsize=None)
def _segsum_call(gpw):
    return pl.kernel(
        functools.partial(_segsum_body, gpw),
        out_type=jax.ShapeDtypeStruct((NP, D), jnp.float32),
        mesh=_mesh(),
        compiler_params=pltpu.CompilerParams(use_tc_tiling_on_sc=False),
        scratch_types=[
            pltpu.VMEM((32, 128), jnp.int32),
            pltpu.VMEM((32, 128), jnp.int32),
            pltpu.VMEM((128, 64), jnp.float32),
            pltpu.VMEM((64, 64), jnp.float32),
            pltpu.VMEM_SHARED((NP, 64), jnp.float32),
            pltpu.VMEM_SHARED((NP, 64), jnp.float32),
            pltpu.SemaphoreType.DMA,
        ],
    )


def _mlp_body(h_ref, agg_ref, w1_ref, b1_ref, w2_ref, b2_ref,
              ga_ref, be_ref, o_ref):
    z = h_ref[...] + agg_ref[...]
    z = jnp.maximum(
        jnp.dot(z, w1_ref[...], preferred_element_type=jnp.float32)
        + b1_ref[...], 0.0)
    z = jnp.maximum(
        jnp.dot(z, w2_ref[...], preferred_element_type=jnp.float32)
        + b2_ref[...], 0.0)
    msk = (lax.broadcasted_iota(jnp.int32, (NP, 1), 0) < N).astype(jnp.float32)
    zm = z * msk
    mean = jnp.sum(zm, axis=0, keepdims=True) * (1.0 / N)
    diff = (z - mean) * msk
    var = jnp.sum(diff * diff, axis=0, keepdims=True) * (1.0 / N)
    o_ref[...] = ((z - mean) * lax.rsqrt(var + 1e-5) * ga_ref[...]
                  + be_ref[...])


def _mlp(h, agg, w1, b1, w2, b2, ga, be):
    return pl.pallas_call(
        _mlp_body,
        out_shape=jax.ShapeDtypeStruct((NP, D), jnp.float32),
    )(h, agg, w1, b1, w2, b2, ga, be)


def _pool_body(z1_hbm, z2_hbm, z3_hbm, bt_hbm, out_hbm,
               bt_v, acc_v, b1_v, b2_v, b3_v, sem):
    c = lax.axis_index("c")
    s = lax.axis_index("s")
    t = s * 2 + c
    glo = t * 8
    ghi = glo + 8

    pltpu.sync_copy(bt_hbm, bt_v.at[pl.ds(0, NP)])
    bt_v[pl.ds(NP, 16)] = jnp.full((16,), 2 ** 30, jnp.int32)

    def lower_bound(val):
        def body(i, st):
            lo, hi = st
            mid = (lo + hi) // 2
            v = bt_v[pl.ds(mid, 16)][0]
            pred = v < val
            return (jnp.where(pred, mid + 1, lo), jnp.where(pred, hi, mid))

        return lax.fori_loop(0, 14, body, (jnp.int32(0), jnp.int32(NP)))[0]

    s0 = lower_bound(glo)
    e0 = lower_bound(ghi)

    neg = jnp.full((16,), -jnp.inf, dtype=jnp.float32)

    def ini(i, carry):
        acc_v[i // 24, pl.ds((i % 24) * 16, 16)] = neg
        return carry

    lax.fori_loop(0, 8 * 24, ini, 0)

    b0 = s0 // 16
    nb = (e0 + 15) // 16 - b0

    def blk(b, carry):
        r0 = pl.multiple_of((b0 + b) * 16, 16)
        pltpu.sync_copy(z1_hbm.at[pl.ds(r0, 16), :], b1_v)
        pltpu.sync_copy(z2_hbm.at[pl.ds(r0, 16), :], b2_v)
        pltpu.sync_copy(z3_hbm.at[pl.ds(r0, 16), :], b3_v)
        bv = bt_v[pl.ds(r0, 16)]

        for k in range(16):
            r = r0 + k

            @pl.when((r >= s0) & (r < e0))
            def _(k=k):
                g = bv[k] - glo

                def feat(j, c3, buf, base):
                    off = base + j * 16
                    acc_v[g, pl.ds(off, 16)] = jnp.maximum(
                        acc_v[g, pl.ds(off, 16)], buf[k, pl.ds(j * 16, 16)])
                    return c3

                lax.fori_loop(0, 8, functools.partial(
                    feat, buf=b1_v, base=0), 0)
                lax.fori_loop(0, 8, functools.partial(
                    feat, buf=b2_v, base=128), 0)
                lax.fori_loop(0, 8, functools.partial(
                    feat, buf=b3_v, base=256), 0)

        return carry

    lax.fori_loop(0, nb, blk, 0)

    pltpu.sync_copy(acc_v, out_hbm.at[pl.ds(glo, 8), :])


@functools.lru_cache(maxsize=None)
def _pool_call():
    return pl.kernel(
        _pool_body,
        out_type=jax.ShapeDtypeStruct((G, L * D), jnp.float32),
        mesh=_mesh(),
        scratch_types=[
            pltpu.VMEM((NP + 16,), jnp.int32),
            pltpu.VMEM((8, L * D), jnp.float32),
            pltpu.VMEM((16, D), jnp.float32),
            pltpu.VMEM((16, D), jnp.float32),
            pltpu.VMEM((16, D), jnp.float32),
            pltpu.SemaphoreType.DMA,
        ],
    )


def kernel(x, edge_index, batch, emb, W1, b1, W2, b2, gamma, beta):
    idx = x[:, 0].astype(jnp.int32)
    n = idx.shape[0]
    idx_p = jnp.pad(idx, (0, NP - n)).reshape(NGROUPS, 128)
    bt_p = jnp.pad(batch.astype(jnp.int32), (0, NP - n), constant_values=G)

    src = edge_index[0].astype(jnp.int32)
    dst = edge_index[1].astype(jnp.int32)
    e = src.shape[0]
    egroups = -(-e // 128)
    gpw = -(-egroups // 16)
    gpw = -(-gpw // 32) * 32
    ep = gpw * 16 * 128
    src_p = jnp.pad(src, (0, ep - e)).reshape(gpw * 16, 128)
    dst_p = jnp.pad(dst, (0, ep - e), constant_values=NP - 1).reshape(
        gpw * 16, 128)

    h = _emb_call()(idx_p, emb)

    outs = []
    for i in range(L):
        agg = _segsum_call(gpw)(h, src_p, dst_p)
        h = _mlp(h, agg, W1[i], b1[i].reshape(1, D), W2[i], b2[i].reshape(1, D),
                 gamma[i].reshape(1, D), beta[i].reshape(1, D))
        outs.append(h)

    return _pool_call()(outs[0], outs[1], outs[2], bt_p)


# speedup vs baseline: 8.0540x; 1.2570x over previous
import functools

import jax
import jax.numpy as jnp
from jax import lax
from jax.experimental import pallas as pl
from jax.experimental.pallas import tpu as pltpu
from jax.experimental.pallas import tpu_sc as plsc

N = 10000
D = 128
G = 256
L = 3
NP = 10240
NGROUPS = NP // 128


def _mesh():
    return plsc.VectorSubcoreMesh(core_axis_name="c", subcore_axis_name="s")


def _emb_body(idx_hbm, emb_hbm, h_hbm, idx_v, rows_v, sem):
    c = lax.axis_index("c")
    s = lax.axis_index("s")
    w = s * 2 + c

    def do(j, carry):
        gid = w + 32 * j

        @pl.when(gid < NGROUPS)
        def _():
            pltpu.sync_copy(idx_hbm.at[gid], idx_v)
            pltpu.async_copy(emb_hbm.at[idx_v], rows_v, sem).wait()
            pltpu.sync_copy(rows_v, h_hbm.at[pl.ds(gid * 128, 128), :])

        return carry

    lax.fori_loop(0, (NGROUPS + 31) // 32, do, 0)


@functools.lru_cache(maxsize=None)
def _emb_call():
    return pl.kernel(
        _emb_body,
        out_type=jax.ShapeDtypeStruct((NP, D), jnp.float32),
        mesh=_mesh(),
        scratch_types=[
            pltpu.VMEM((128,), jnp.int32),
            pltpu.VMEM((128, D), jnp.float32),
            pltpu.SemaphoreType.DMA,
        ],
    )


def _segsum_body(gpw, h_hbm, src_hbm, dst_hbm, agg_hbm,
                 src_v, dst_v, rows_a, rows_b, zed_v, h_sp, agg_sp,
                 sem_a, sem_b):
    c = lax.axis_index("c")
    s = lax.axis_index("s")
    col0 = c * 64
    row0 = s * (NP // 16)

    pltpu.sync_copy(h_hbm.at[pl.ds(row0, NP // 16), pl.ds(col0, 64)],
                    h_sp.at[pl.ds(row0, NP // 16), :])

    def zstore(i, carry):
        zed_v[i // 4, pl.ds((i % 4) * 16, 16)] = jnp.zeros((16,), jnp.float32)
        return carry

    lax.fori_loop(0, 64 * 4, zstore, 0)

    def zcp(i, carry):
        pltpu.sync_copy(zed_v, agg_sp.at[pl.ds(row0 + i * 64, 64), :])
        return carry

    lax.fori_loop(0, (NP // 16) // 64, zcp, 0)

    plsc.subcore_barrier()

    g0 = s * gpw

    def chunk(ci, carry):
        cg = g0 + ci * 32
        pltpu.sync_copy(src_hbm.at[pl.ds(cg, 32), :], src_v)
        pltpu.sync_copy(dst_hbm.at[pl.ds(cg, 32), :], dst_v)

        pltpu.async_copy(h_sp.at[src_v.at[0]], rows_a, sem_a)

        def pair(jj, carry2):
            j0 = jj * 2
            j1 = j0 + 1
            pltpu.async_copy(h_sp.at[src_v.at[j1]], rows_b, sem_b)
            pltpu.make_async_copy(h_sp.at[src_v.at[j0]], rows_a, sem_a).wait()
            pltpu.sync_copy(rows_a, agg_sp.at[dst_v.at[j0]], add=True)

            @pl.when(jj < 15)
            def _():
                pltpu.async_copy(h_sp.at[src_v.at[j0 + 2]], rows_a, sem_a)

            pltpu.make_async_copy(h_sp.at[src_v.at[j1]], rows_b, sem_b).wait()
            pltpu.sync_copy(rows_b, agg_sp.at[dst_v.at[j1]], add=True)
            return carry2

        lax.fori_loop(0, 16, pair, 0)
        return carry

    lax.fori_loop(0, gpw // 32, chunk, 0)

    plsc.subcore_barrier()
    pltpu.sync_copy(agg_sp.at[pl.ds(row0, NP // 16), :],
                    agg_hbm.at[pl.ds(row0, NP // 16), pl.ds(col0, 64)])


@functools.lru_cache(maxsize=None)
def _segsum_call(gpw):
    return pl.kernel(
        functools.partial(_segsum_body, gpw),
        out_type=jax.ShapeDtypeStruct((NP, D), jnp.float32),
        mesh=_mesh(),
        compiler_params=pltpu.CompilerParams(use_tc_tiling_on_sc=False),
        scratch_types=[
            pltpu.VMEM((32, 128), jnp.int32),
            pltpu.VMEM((32, 128), jnp.int32),
            pltpu.VMEM((128, 64), jnp.float32),
            pltpu.VMEM((128, 64), jnp.float32),
            pltpu.VMEM((64, 64), jnp.float32),
            pltpu.VMEM_SHARED((NP, 64), jnp.float32),
            pltpu.VMEM_SHARED((NP, 64), jnp.float32),
            pltpu.SemaphoreType.DMA,
            pltpu.SemaphoreType.DMA,
        ],
    )


def _mlp_body(h_ref, agg_ref, w1_ref, b1_ref, w2_ref, b2_ref,
              ga_ref, be_ref, o_ref):
    z = h_ref[...] + agg_ref[...]
    z = jnp.maximum(
        jnp.dot(z, w1_ref[...], preferred_element_type=jnp.float32)
        + b1_ref[...], 0.0)
    z = jnp.maximum(
        jnp.dot(z, w2_ref[...], preferred_element_type=jnp.float32)
        + b2_ref[...], 0.0)
    msk = (lax.broadcasted_iota(jnp.int32, (NP, 1), 0) < N).astype(jnp.float32)
    zm = z * msk
    mean = jnp.sum(zm, axis=0, keepdims=True) * (1.0 / N)
    diff = (z - mean) * msk
    var = jnp.sum(diff * diff, axis=0, keepdims=True) * (1.0 / N)
    o_ref[...] = ((z - mean) * lax.rsqrt(var + 1e-5) * ga_ref[...]
                  + be_ref[...])


def _mlp(h, agg, w1, b1, w2, b2, ga, be):
    return pl.pallas_call(
        _mlp_body,
        out_shape=jax.ShapeDtypeStruct((NP, D), jnp.float32),
    )(h, agg, w1, b1, w2, b2, ga, be)


def _pool_body(z1_hbm, z2_hbm, z3_hbm, bt_hbm, out_hbm,
               bt_v, acc_v, b1_v, b2_v, b3_v, sem):
    c = lax.axis_index("c")
    s = lax.axis_index("s")
    t = s * 2 + c
    glo = t * 8
    ghi = glo + 8

    pltpu.sync_copy(bt_hbm, bt_v.at[pl.ds(0, NP)])
    bt_v[pl.ds(NP, 16)] = jnp.full((16,), 2 ** 30, jnp.int32)

    def lower_bound(val):
        def body(i, st):
            lo, hi = st
            mid = (lo + hi) // 2
            v = bt_v[pl.ds(mid, 16)][0]
            pred = v < val
            return (jnp.where(pred, mid + 1, lo), jnp.where(pred, hi, mid))

        return lax.fori_loop(0, 14, body, (jnp.int32(0), jnp.int32(NP)))[0]

    s0 = lower_bound(glo)
    e0 = lower_bound(ghi)

    neg = jnp.full((16,), -jnp.inf, dtype=jnp.float32)

    def ini(i, carry):
        acc_v[i // 24, pl.ds((i % 24) * 16, 16)] = neg
        return carry

    lax.fori_loop(0, 8 * 24, ini, 0)

    b0 = s0 // 16
    nb = (e0 + 15) // 16 - b0

    def blk(b, carry):
        r0 = pl.multiple_of((b0 + b) * 16, 16)
        pltpu.sync_copy(z1_hbm.at[pl.ds(r0, 16), :], b1_v)
        pltpu.sync_copy(z2_hbm.at[pl.ds(r0, 16), :], b2_v)
        pltpu.sync_copy(z3_hbm.at[pl.ds(r0, 16), :], b3_v)
        bv = bt_v[pl.ds(r0, 16)]

        for k in range(16):
            r = r0 + k

            @pl.when((r >= s0) & (r < e0))
            def _(k=k):
                g = bv[k] - glo

                def feat(j, c3, buf, base):
                    off = base + j * 16
                    acc_v[g, pl.ds(off, 16)] = jnp.maximum(
                        acc_v[g, pl.ds(off, 16)], buf[k, pl.ds(j * 16, 16)])
                    return c3

                lax.fori_loop(0, 8, functools.partial(
                    feat, buf=b1_v, base=0), 0)
                lax.fori_loop(0, 8, functools.partial(
                    feat, buf=b2_v, base=128), 0)
                lax.fori_loop(0, 8, functools.partial(
                    feat, buf=b3_v, base=256), 0)

        return carry

    lax.fori_loop(0, nb, blk, 0)

    pltpu.sync_copy(acc_v, out_hbm.at[pl.ds(glo, 8), :])


@functools.lru_cache(maxsize=None)
def _pool_call():
    return pl.kernel(
        _pool_body,
        out_type=jax.ShapeDtypeStruct((G, L * D), jnp.float32),
        mesh=_mesh(),
        scratch_types=[
            pltpu.VMEM((NP + 16,), jnp.int32),
            pltpu.VMEM((8, L * D), jnp.float32),
            pltpu.VMEM((16, D), jnp.float32),
            pltpu.VMEM((16, D), jnp.float32),
            pltpu.VMEM((16, D), jnp.float32),
            pltpu.SemaphoreType.DMA,
        ],
    )


def kernel(x, edge_index, batch, emb, W1, b1, W2, b2, gamma, beta):
    idx = x[:, 0].astype(jnp.int32)
    n = idx.shape[0]
    idx_p = jnp.pad(idx, (0, NP - n)).reshape(NGROUPS, 128)
    bt_p = jnp.pad(batch.astype(jnp.int32), (0, NP - n), constant_values=G)

    src = edge_index[0].astype(jnp.int32)
    dst = edge_index[1].astype(jnp.int32)
    e = src.shape[0]
    egroups = -(-e // 128)
    gpw = -(-egroups // 16)
    gpw = -(-gpw // 32) * 32
    ep = gpw * 16 * 128
    src_p = jnp.pad(src, (0, ep - e)).reshape(gpw * 16, 128)
    dst_p = jnp.pad(dst, (0, ep - e), constant_values=NP - 1).reshape(
        gpw * 16, 128)

    h = _emb_call()(idx_p, emb)

    outs = []
    for i in range(L):
        agg = _segsum_call(gpw)(h, src_p, dst_p)
        h = _mlp(h, agg, W1[i], b1[i].reshape(1, D), W2[i], b2[i].reshape(1, D),
                 gamma[i].reshape(1, D), beta[i].reshape(1, D))
        outs.append(h)

    return _pool_call()(outs[0], outs[1], outs[2], bt_p)
